# Initial kernel scaffold; baseline (speedup 1.0000x reference)
import jax
import jax.numpy as jnp
from jax.experimental import pallas as pl


def kernel(nums, emb):
    raise NotImplementedError("write your pallas kernel here")


# speedup vs baseline: 4.5988x; 4.5988x over previous
import functools

import jax
import jax.numpy as jnp
from jax import lax
from jax.experimental import pallas as pl
from jax.experimental.pallas import tpu as pltpu
from jax.experimental.pallas import tpu_sc as plsc

DIGITS = 8
HIDDEN = 128
BATCH = 16384
NLANES = 16
NCORES = 2
NSUB = 16
NW = NCORES * NSUB
BPW = BATCH // NW
HREGS = HIDDEN // NLANES


def _sc_body(nums_hbm, emb_hbm, out_hbm, emb_v, nums_v, out_v):
    wid = lax.axis_index("s") * NCORES + lax.axis_index("c")
    base = wid * BPW
    pltpu.sync_copy(emb_hbm, emb_v)
    pltpu.sync_copy(nums_hbm.at[pl.ds(base, BPW)], nums_v)

    def body(g, carry):
        n = nums_v[pl.ds(g * NLANES, NLANES)]
        rows = []
        for i in range(DIGITS):
            rows.append(lax.rem(n, 10) + 10 * i)
            n = lax.div(n, 10)
        for k in range(NLANES):
            accs = [None] * HREGS
            for i in range(DIGITS):
                d = rows[i][k]
                for h in range(HREGS):
                    v = emb_v[d, pl.ds(h * NLANES, NLANES)]
                    accs[h] = v if i == 0 else accs[h] + v
            j = g * NLANES + k
            for h in range(HREGS):
                out_v[j, pl.ds(h * NLANES, NLANES)] = accs[h]
        return carry

    lax.fori_loop(0, BPW // NLANES, body, 0)
    pltpu.sync_copy(out_v, out_hbm.at[pl.ds(base, BPW)])


@functools.partial(jax.jit, static_argnames=())
def kernel(nums, emb):
    nums = nums.astype(jnp.int32)
    mesh = plsc.VectorSubcoreMesh(core_axis_name="c", subcore_axis_name="s")
    f = functools.partial(
        pl.kernel,
        out_type=jax.ShapeDtypeStruct((BATCH, HIDDEN), jnp.float32),
        mesh=mesh,
        scratch_types=[
            pltpu.VMEM((DIGITS * 10, HIDDEN), jnp.float32),
            pltpu.VMEM((BPW,), jnp.int32),
            pltpu.VMEM((BPW, HIDDEN), jnp.float32),
        ],
    )(_sc_body)
    return f(nums, emb)
